# Initial kernel scaffold; baseline (speedup 1.0000x reference)
import jax
import jax.numpy as jnp
from jax.experimental import pallas as pl


def kernel(x, W1, b1, W2, b2, We, be, Dw2, Dw1, Dpw, r, noise):
    raise NotImplementedError("write your pallas kernel here")


# speedup vs baseline: 4.4480x; 4.4480x over previous
import functools

import jax
import jax.numpy as jnp
from jax import lax
from jax.experimental import pallas as pl
from jax.experimental.pallas import tpu as pltpu

_B = 8192
_BINS = 300
_D = 20
_HID = 50
_NC = 2
_BA = 256
_BB = 1024


def _wconv(l, r, pad):
    return lax.conv_general_dilated(
        l, r, (1, 1), padding=pad,
        dimension_numbers=('NCHW', 'OIHW', 'NCHW'),
        precision=lax.Precision.HIGHEST)


def _fold_chain(Wa, Wb, Wlin):
    Waf = Wa[:, :, ::-1, ::-1].transpose(1, 0, 2, 3)
    Weff = _wconv(Wb, Waf, ((9, 9), (9, 9)))
    WlinR = Wlin.reshape(_D, Wb.shape[0], 10, 10)
    Wefff = Weff[:, :, ::-1, ::-1].transpose(1, 0, 2, 3)
    M4 = _wconv(WlinR, Wefff, ((18, 18), (18, 18)))
    return M4.reshape(_D, 28 * 28)


def _enc_body(x_ref, w1c_ref, b1r_ref, w2c_ref, b2r_ref, wep_ref, be_ref,
              md_ref, p_ref, xi_ref, yj_ref, f_ref, h1_ref, h2_ref):
    j = pl.program_id(1)
    xb16 = x_ref[...]
    dn = (((1,), (1,)), ((), ()))

    for rr in range(19):
        h1r = lax.dot_general(xb16[:, 28 * rr:28 * rr + 280], w1c_ref[...],
                              dn, preferred_element_type=jnp.float32)
        h1r = h1r + b1r_ref[...]
        h1_ref[:, 380 * rr:380 * (rr + 1)] = h1r.astype(jnp.bfloat16)
    for pp in range(10):
        h2r = lax.dot_general(h1_ref[:, 380 * pp:380 * pp + 3800],
                              w2c_ref[...], dn,
                              preferred_element_type=jnp.float32)
        h2r = h2r + b2r_ref[...]
        h2_ref[:, 500 * pp:500 * (pp + 1)] = h2r.astype(jnp.bfloat16)
    xi = lax.dot_general(wep_ref[...], h2_ref[...], dn,
                         preferred_element_type=jnp.float32) + be_ref[...]

    yj = lax.dot_general(md_ref[...], xb16.astype(jnp.float32), dn,
                         preferred_element_type=jnp.float32,
                         precision=lax.Precision.HIGHEST)
    xi_ref[...] = xi
    yj_ref[...] = yj
    t = p_ref[...][None, :, :] - xi[:, None, :]
    fc = jnp.exp(-(t * t)).sum(axis=2)

    @pl.when(j == 0)
    def _():
        f_ref[0] = fc

    @pl.when(j > 0)
    def _():
        f_ref[0] += fc


def _cdf_body(f_ref, cdf_ref):
    f = f_ref[0] + f_ref[1]
    fn = f / jnp.sum(f, axis=1, keepdims=True)
    ri = lax.broadcasted_iota(jnp.int32, (_BINS, _BINS), 0)
    ci = lax.broadcasted_iota(jnp.int32, (_BINS, _BINS), 1)
    tri = jnp.where(ri <= ci, 1.0, 0.0)
    cdf = jnp.dot(fn, tri, preferred_element_type=jnp.float32,
                  precision=lax.Precision.HIGHEST)
    cdf_ref[...] = cdf[:, :, None]


def _sample_body(r_ref, cdf_ref, p_ref, n_ref, xi_ref, pp_ref, yj_ref):
    rb = r_ref[0]
    dv = jnp.abs(cdf_ref[0] - rb)
    pv = jnp.min(dv, axis=0)
    idx = jnp.argmin(dv, axis=0)
    ii = lax.broadcasted_iota(jnp.int32, (_BINS, _BB), 0)
    xv = jnp.sum(jnp.where(ii == idx[None, :], p_ref[...], 0.0), axis=0)
    xi_ref[0, 0, :] = xv
    pp_ref[0, 0, :] = pv
    yj_ref[0] = xv[None, :] + n_ref[0]


@functools.partial(jax.jit, static_argnums=())
def kernel(x, W1, b1, W2, b2, We, be, Dw2, Dw1, Dpw, r, noise):
    W1b = W1.astype(jnp.bfloat16)
    W2b = W2.astype(jnp.bfloat16)
    Web = We.astype(jnp.bfloat16)
    w1p = jnp.stack([jnp.pad(W1b[:, 0], ((0, 0), (0, 0), (s, 18 - s)))
                     for s in range(19)], axis=1)
    W1c = w1p.reshape(380, 280)
    b1row = jnp.broadcast_to(b1[:, None], (_D, 19)).reshape(1, 380)
    w2p = jnp.stack([jnp.pad(W2b, ((0, 0), (0, 0), (0, 0), (q, 9 - q)))
                     for q in range(10)], axis=1)
    W2c = w2p.transpose(0, 1, 3, 2, 4).reshape(500, 3800)
    b2row = jnp.broadcast_to(b2[:, None], (_HID, 10)).reshape(1, 500)
    We_perm = Web.reshape(_D, _HID, 10, 10).transpose(0, 2, 1, 3).reshape(_D, 5000)
    be_col = be.reshape(_D, 1)

    def rdw(a):
        return a.astype(jnp.bfloat16).astype(jnp.float32)
    Mdec = _fold_chain(rdw(Dw2), rdw(Dw1), rdw(Dpw.T.reshape(_D, 5000)))
    xf = x.astype(jnp.bfloat16).reshape(_B, 28 * 28)
    pcol = jnp.linspace(-100.0, 100.0, _BINS, dtype=jnp.float32).reshape(_BINS, 1)

    nja = _B // (_NC * _BA)
    xi, yj, fpart = pl.pallas_call(
        _enc_body,
        grid=(_NC, nja),
        in_specs=[
            pl.BlockSpec((_BA, 784), lambda c, j: (c * nja + j, 0)),
            pl.BlockSpec((380, 280), lambda c, j: (0, 0)),
            pl.BlockSpec((1, 380), lambda c, j: (0, 0)),
            pl.BlockSpec((500, 3800), lambda c, j: (0, 0)),
            pl.BlockSpec((1, 500), lambda c, j: (0, 0)),
            pl.BlockSpec((_D, 5000), lambda c, j: (0, 0)),
            pl.BlockSpec((_D, 1), lambda c, j: (0, 0)),
            pl.BlockSpec((_D, 784), lambda c, j: (0, 0)),
            pl.BlockSpec((_BINS, 1), lambda c, j: (0, 0)),
        ],
        out_specs=[
            pl.BlockSpec((_D, _BA), lambda c, j: (0, c * nja + j)),
            pl.BlockSpec((_D, _BA), lambda c, j: (0, c * nja + j)),
            pl.BlockSpec((1, _D, _BINS), lambda c, j: (c, 0, 0)),
        ],
        out_shape=[
            jax.ShapeDtypeStruct((_D, _B), jnp.float32),
            jax.ShapeDtypeStruct((_D, _B), jnp.float32),
            jax.ShapeDtypeStruct((_NC, _D, _BINS), jnp.float32),
        ],
        scratch_shapes=[
            pltpu.VMEM((_BA, 7220), jnp.bfloat16),
            pltpu.VMEM((_BA, 5000), jnp.bfloat16),
        ],
        compiler_params=pltpu.CompilerParams(
            dimension_semantics=("parallel", "arbitrary")),
    )(xf, W1c, b1row, W2c, b2row, We_perm, be_col, Mdec, pcol)

    cdf3 = pl.pallas_call(
        _cdf_body,
        in_specs=[pl.BlockSpec((_NC, _D, _BINS), lambda: (0, 0, 0))],
        out_specs=pl.BlockSpec((_D, _BINS, 1), lambda: (0, 0, 0)),
        out_shape=jax.ShapeDtypeStruct((_D, _BINS, 1), jnp.float32),
    )(fpart)

    njb = _B // (_NC * _BB)
    xi3, p3, yj_ = pl.pallas_call(
        _sample_body,
        grid=(_NC, _D, njb),
        in_specs=[
            pl.BlockSpec((1, _BINS, _BB), lambda c, d, j: (d, 0, c * njb + j)),
            pl.BlockSpec((1, _BINS, 1), lambda c, d, j: (d, 0, 0)),
            pl.BlockSpec((_BINS, 1), lambda c, d, j: (0, 0)),
            pl.BlockSpec((1, 10, _BB), lambda c, d, j: (d, 0, c * njb + j)),
        ],
        out_specs=[
            pl.BlockSpec((1, 1, _BB), lambda c, d, j: (d, 0, c * njb + j)),
            pl.BlockSpec((1, 1, _BB), lambda c, d, j: (d, 0, c * njb + j)),
            pl.BlockSpec((1, 10, _BB), lambda c, d, j: (d, 0, c * njb + j)),
        ],
        out_shape=[
            jax.ShapeDtypeStruct((_D, 1, _B), jnp.float32),
            jax.ShapeDtypeStruct((_D, 1, _B), jnp.float32),
            jax.ShapeDtypeStruct((_D, 10, _B), jnp.float32),
        ],
        compiler_params=pltpu.CompilerParams(
            dimension_semantics=("parallel", "arbitrary", "arbitrary")),
    )(r, cdf3, pcol, noise)

    return xi, yj, xi3.reshape(_D, _B), yj_, p3.reshape(_D, _B)
